# baseline (speedup 1.0000x reference)
import functools

import jax
import jax.numpy as jnp
from jax import lax
from jax.experimental import pallas as pl
from jax.experimental.pallas import tpu as pltpu
from jax.experimental.pallas import tpu_sc as plsc

VOCAB = 1000000
EMBED = 64
OUT = 100000
B = 1024
L = 200

NC = 2
NS = 16
NW = NC * NS
BPW = B // NW
NG = L // 16
REM = L - NG * 16


def _sc_pool_body(idx_hbm, table_hbm, out_hbm, idx_v, rows_v, acc_v, sem):
    wid = lax.axis_index("s") * NC + lax.axis_index("c")
    base = wid * BPW
    pltpu.sync_copy(idx_hbm.at[pl.ds(base, BPW)], idx_v)
    lanes = lax.iota(jnp.int32, 16)

    def extract(vec, l):
        return jnp.sum(jnp.where(lanes == l, vec, 0))

    def row_body(i, carry_unused):
        def fire_group(g, _):
            vec = idx_v[i, pl.ds(g * 16, 16)]
            for l in range(16):
                r = extract(vec, l)
                pltpu.async_copy(
                    table_hbm.at[pl.ds(r, 1)],
                    rows_v.at[pl.ds(g * 16 + l, 1)], sem)
            return 0

        lax.fori_loop(0, NG, fire_group, 0)
        vec = idx_v[i, pl.ds(L - 16, 16)]
        for l in range(16 - REM, 16):
            r = extract(vec, l)
            pltpu.async_copy(
                table_hbm.at[pl.ds(r, 1)],
                rows_v.at[pl.ds(L - 16 + l, 1)], sem)
        pltpu.make_async_copy(table_hbm.at[pl.ds(0, L)], rows_v, sem).wait()

        def acc_body(j, carry):
            a0, a1, a2, a3 = carry
            a0 = a0 + rows_v[j, pl.ds(0, 16)]
            a1 = a1 + rows_v[j, pl.ds(16, 16)]
            a2 = a2 + rows_v[j, pl.ds(32, 16)]
            a3 = a3 + rows_v[j, pl.ds(48, 16)]
            return a0, a1, a2, a3

        z = jnp.zeros((16,), jnp.float32)
        a0, a1, a2, a3 = lax.fori_loop(0, L, acc_body, (z, z, z, z))
        acc_v[i, pl.ds(0, 16)] = a0
        acc_v[i, pl.ds(16, 16)] = a1
        acc_v[i, pl.ds(32, 16)] = a2
        acc_v[i, pl.ds(48, 16)] = a3
        return 0

    lax.fori_loop(0, BPW, row_body, 0)
    pltpu.sync_copy(acc_v, out_hbm.at[pl.ds(base, BPW)])


_sc_pool = functools.partial(
    pl.kernel,
    mesh=plsc.VectorSubcoreMesh(core_axis_name="c", subcore_axis_name="s"),
    out_type=jax.ShapeDtypeStruct((B, EMBED), jnp.float32),
    scratch_types=[
        pltpu.VMEM((BPW, L), jnp.int32),
        pltpu.VMEM((L, EMBED), jnp.float32),
        pltpu.VMEM((BPW, EMBED), jnp.float32),
        pltpu.SemaphoreType.DMA,
    ],
    compiler_params=pltpu.CompilerParams(needs_layout_passes=False),
)(_sc_pool_body)


BLK = 1024
NBLK = (OUT + BLK - 1) // BLK
K1 = EMBED + 1


def _mm_body(wt_ref, p_ref, o_ref):
    o_ref[:] = lax.dot_general(
        wt_ref[:], p_ref[:], (((0,), (1,)), ((), ())),
        preferred_element_type=jnp.float32)


def _matmul_t(Wbt, pooled1):
    return pl.pallas_call(
        _mm_body,
        grid=(NBLK,),
        in_specs=[
            pl.BlockSpec((K1, BLK), lambda j: (0, j)),
            pl.BlockSpec((B, K1), lambda j: (0, 0)),
        ],
        out_specs=pl.BlockSpec((BLK, B), lambda j: (j, 0)),
        out_shape=jax.ShapeDtypeStruct((OUT, B), jnp.float32),
    )(Wbt, pooled1)


def kernel(inputs, table, W, b):
    pooled = _sc_pool(inputs.astype(jnp.int32), table)
    pooled1 = jnp.concatenate([pooled, jnp.ones((B, 1), jnp.float32)], axis=1)
    Wbt = jnp.concatenate([W, b[:, None]], axis=1).T
    out_t = _matmul_t(Wbt, pooled1)
    return out_t.T


# speedup vs baseline: 1.0410x; 1.0410x over previous
import functools

import jax
import jax.numpy as jnp
from jax import lax
from jax.experimental import pallas as pl
from jax.experimental.pallas import tpu as pltpu
from jax.experimental.pallas import tpu_sc as plsc

VOCAB = 1000000
EMBED = 64
OUT = 100000
B = 1024
L = 200

NC = 2
NS = 16
NW = NC * NS
BPW = B // NW
NG = L // 16
REM = L - NG * 16


def _sc_pool_body(idx_hbm, table_hbm, out_hbm, idx_v, rows_v, acc_v, sem):
    wid = lax.axis_index("s") * NC + lax.axis_index("c")
    base = wid * BPW
    pltpu.sync_copy(idx_hbm.at[pl.ds(base, BPW)], idx_v)
    lanes = lax.iota(jnp.int32, 16)

    def extract(vec, l):
        return jnp.sum(jnp.where(lanes == l, vec, 0))

    def row_body(i, carry_unused):
        def fire_group(g, _):
            vec = idx_v[i, pl.ds(g * 16, 16)]
            for l in range(16):
                r = extract(vec, l)
                pltpu.async_copy(
                    table_hbm.at[pl.ds(r, 1)],
                    rows_v.at[pl.ds(g * 16 + l, 1)], sem)
            return 0

        lax.fori_loop(0, NG, fire_group, 0)
        vec = idx_v[i, pl.ds(L - 16, 16)]
        for l in range(16 - REM, 16):
            r = extract(vec, l)
            pltpu.async_copy(
                table_hbm.at[pl.ds(r, 1)],
                rows_v.at[pl.ds(L - 16 + l, 1)], sem)
        pltpu.make_async_copy(table_hbm.at[pl.ds(0, L)], rows_v, sem).wait()

        def acc_body(j, carry):
            a0, a1, a2, a3 = carry
            a0 = a0 + rows_v[j, pl.ds(0, 16)]
            a1 = a1 + rows_v[j, pl.ds(16, 16)]
            a2 = a2 + rows_v[j, pl.ds(32, 16)]
            a3 = a3 + rows_v[j, pl.ds(48, 16)]
            return a0, a1, a2, a3

        z = jnp.zeros((16,), jnp.float32)
        a0, a1, a2, a3 = lax.fori_loop(0, L, acc_body, (z, z, z, z))
        acc_v[i, pl.ds(0, 16)] = a0
        acc_v[i, pl.ds(16, 16)] = a1
        acc_v[i, pl.ds(32, 16)] = a2
        acc_v[i, pl.ds(48, 16)] = a3
        return 0

    lax.fori_loop(0, BPW, row_body, 0)
    pltpu.sync_copy(acc_v, out_hbm.at[pl.ds(base, BPW)])


_sc_pool = functools.partial(
    pl.kernel,
    mesh=plsc.VectorSubcoreMesh(core_axis_name="c", subcore_axis_name="s"),
    out_type=jax.ShapeDtypeStruct((B, EMBED), jnp.float32),
    scratch_types=[
        pltpu.VMEM((BPW, L), jnp.int32),
        pltpu.VMEM((L, EMBED), jnp.float32),
        pltpu.VMEM((BPW, EMBED), jnp.float32),
        pltpu.SemaphoreType.DMA,
    ],
    compiler_params=pltpu.CompilerParams(needs_layout_passes=False),
)(_sc_pool_body)


BLK = 4096
NBLK = (OUT + BLK - 1) // BLK
K1 = EMBED + 1


def _mm_body(wt_ref, p_ref, o_ref):
    o_ref[:] = lax.dot_general(
        wt_ref[:], p_ref[:], (((0,), (1,)), ((), ())),
        preferred_element_type=jnp.float32)


def _matmul_t(Wbt, pooled1):
    return pl.pallas_call(
        _mm_body,
        grid=(NBLK,),
        in_specs=[
            pl.BlockSpec((K1, BLK), lambda j: (0, j)),
            pl.BlockSpec((B, K1), lambda j: (0, 0)),
        ],
        out_specs=pl.BlockSpec((BLK, B), lambda j: (j, 0)),
        out_shape=jax.ShapeDtypeStruct((OUT, B), jnp.float32),
    )(Wbt, pooled1)


def kernel(inputs, table, W, b):
    pooled = _sc_pool(inputs.astype(jnp.int32), table)
    pooled1 = jnp.concatenate([pooled, jnp.ones((B, 1), jnp.float32)], axis=1)
    Wbt = jnp.concatenate([W, b[:, None]], axis=1).T
    out_t = _matmul_t(Wbt, pooled1)
    return out_t.T
